# baseline (speedup 1.0000x reference)
import jax
import jax.numpy as jnp
import numpy as np
from jax import lax
from jax.experimental import pallas as pl
from jax.experimental.pallas import tpu as pltpu
from jax.experimental.pallas import tpu_sc as plsc

_N = 10000
_E = 320000
_D = 128
_NC = 2
_NS = 16
_NW = _NC * _NS
_NP = 10240
_RPT = _NP // _NS
_EPT_DEG = _E // _NS
_EPT = _E // _NW
_C = 96
_NCHUNK = -(-_EPT // _C)
_EPT2 = _NCHUNK * _C
_ZFULL = _RPT // _C
_ZREM = _RPT - _ZFULL * _C
_R = 1024
_G = _NP // _R
_BN = float(1.0 / np.sqrt(1.0 + 1e-5))


def _dinv_body(dst_hbm, dinv_hbm, dstall, degbuf, tmp, stage_sh):
    c = lax.axis_index("c")
    s = lax.axis_index("s")
    zeros = jnp.zeros((16,), jnp.float32)
    ones = jnp.ones((16,), jnp.float32)

    @pl.when(c == 0)
    def _():
        def zero_step(i, carry):
            degbuf[pl.ds(i * 16, 16)] = zeros
            return carry

        lax.fori_loop(0, _NP // 16, zero_step, 0)

        pltpu.sync_copy(dst_hbm.at[pl.ds(s * _EPT_DEG, _EPT_DEG)], dstall)

        def scat_step(i, carry):
            idx = dstall[pl.ds(i * 16, 16)]
            plsc.addupdate_scatter(degbuf, [idx], ones)
            return carry

        lax.fori_loop(0, _EPT_DEG // 16, scat_step, 0)

        pltpu.sync_copy(degbuf, stage_sh.at[pl.ds(s * _NP, _NP)])
        plsc.subcore_barrier()

        for j in range(_NS):
            pltpu.sync_copy(stage_sh.at[pl.ds(j * _NP + s * _RPT, _RPT)],
                            tmp.at[pl.ds(j * _RPT, _RPT)])

        lane = lax.iota(jnp.int32, 16)

        def red_step(k, carry):
            acc = tmp[pl.ds(k * 16, 16)]
            for j in range(1, _NS):
                acc = acc + tmp[pl.ds(j * _RPT + k * 16, 16)]
            x = acc + 1.0
            ibits = lax.bitcast_convert_type(x, jnp.int32)
            ibits = jnp.int32(0x5F3759DF) - (ibits >> 1)
            y = lax.bitcast_convert_type(ibits, jnp.float32)
            for _ in range(3):
                y = y * (1.5 - 0.5 * x * y * y)
            node = s * _RPT + k * 16 + lane
            y = jnp.where(node < _N, y, 0.0)
            tmp[pl.ds(k * 16, 16)] = y
            return carry

        lax.fori_loop(0, _RPT // 16, red_step, 0)
        pltpu.sync_copy(tmp.at[pl.ds(0, _RPT)],
                        dinv_hbm.at[pl.ds(s * _RPT, _RPT)])


_dinv_call = pl.kernel(
    _dinv_body,
    out_type=jax.ShapeDtypeStruct((_NP,), jnp.float32),
    mesh=plsc.VectorSubcoreMesh(
        core_axis_name="c", subcore_axis_name="s", num_cores=_NC,
        num_subcores=_NS),
    scratch_types=[
        pltpu.VMEM((_EPT_DEG,), jnp.int32),
        pltpu.VMEM((_NP,), jnp.float32),
        pltpu.VMEM((_NS * _RPT,), jnp.float32),
        pltpu.VMEM_SHARED((_NS * _NP,), jnp.float32),
    ],
    compiler_params=pltpu.CompilerParams(needs_layout_passes=False),
)


def _agg_body(hp_hbm, src_hbm, dst_hbm, out_hbm,
              srcall, dstall, srcbuf0, dstbuf0, srcbuf1, dstbuf1,
              rows0, rows1, s_sh, gsem0, gsem1):
    c = lax.axis_index("c")
    s = lax.axis_index("s")
    wid = c * _NS + s
    base = wid * _EPT2
    zeros = jnp.zeros((16,), jnp.float32)

    def zero_step(i, carry):
        rows0[i // 8, pl.ds((i % 8) * 16, 16)] = zeros
        return carry

    lax.fori_loop(0, _C * 8, zero_step, 0)
    for r in range(_ZFULL):
        pltpu.sync_copy(rows0, s_sh.at[pl.ds(s * _RPT + r * _C, _C)])
    if _ZREM:
        pltpu.sync_copy(rows0.at[pl.ds(0, _ZREM)],
                        s_sh.at[pl.ds(s * _RPT + _ZFULL * _C, _ZREM)])
    plsc.subcore_barrier()

    pltpu.sync_copy(src_hbm.at[pl.ds(base, _EPT2)], srcall)
    pltpu.sync_copy(dst_hbm.at[pl.ds(base, _EPT2)], dstall)

    def fill(k, sbuf, dbuf):
        for j in range(_C // 16):
            sbuf[pl.ds(j * 16, 16)] = srcall[pl.ds(k * _C + j * 16, 16)]
            dbuf[pl.ds(j * 16, 16)] = dstall[pl.ds(k * _C + j * 16, 16)]

    fill(0, srcbuf0, dstbuf0)
    pltpu.async_copy(hp_hbm.at[srcbuf0], rows0, gsem0)

    def pair_step(i, carry):
        k = 2 * i
        fill(k + 1, srcbuf1, dstbuf1)
        pltpu.async_copy(hp_hbm.at[srcbuf1], rows1, gsem1)
        pltpu.make_async_copy(hp_hbm.at[srcbuf0], rows0, gsem0).wait()
        pltpu.sync_copy(rows0, s_sh.at[dstbuf0], add=True)
        fill(k + 2, srcbuf0, dstbuf0)
        pltpu.async_copy(hp_hbm.at[srcbuf0], rows0, gsem0)
        pltpu.make_async_copy(hp_hbm.at[srcbuf1], rows1, gsem1).wait()
        pltpu.sync_copy(rows1, s_sh.at[dstbuf1], add=True)
        return carry

    lax.fori_loop(0, (_NCHUNK - 1) // 2, pair_step, 0)
    pltpu.make_async_copy(hp_hbm.at[srcbuf0], rows0, gsem0).wait()
    pltpu.sync_copy(rows0, s_sh.at[dstbuf0], add=True)
    plsc.subcore_barrier()

    for r in range(_ZFULL):
        pltpu.sync_copy(s_sh.at[pl.ds(s * _RPT + r * _C, _C)], rows0)
        pltpu.sync_copy(rows0, out_hbm.at[c, pl.ds(s * _RPT + r * _C, _C)])
    if _ZREM:
        off = s * _RPT + _ZFULL * _C
        pltpu.sync_copy(s_sh.at[pl.ds(off, _ZREM)], rows0.at[pl.ds(0, _ZREM)])
        pltpu.sync_copy(rows0.at[pl.ds(0, _ZREM)],
                        out_hbm.at[c, pl.ds(off, _ZREM)])


_agg_call = pl.kernel(
    _agg_body,
    out_type=jax.ShapeDtypeStruct((_NC, _NP, _D), jnp.float32),
    mesh=plsc.VectorSubcoreMesh(
        core_axis_name="c", subcore_axis_name="s", num_cores=_NC,
        num_subcores=_NS),
    scratch_types=[
        pltpu.VMEM((_EPT2,), jnp.int32),
        pltpu.VMEM((_EPT2,), jnp.int32),
        pltpu.VMEM((_C,), jnp.int32),
        pltpu.VMEM((_C,), jnp.int32),
        pltpu.VMEM((_C,), jnp.int32),
        pltpu.VMEM((_C,), jnp.int32),
        pltpu.VMEM((_C, _D), jnp.float32),
        pltpu.VMEM((_C, _D), jnp.float32),
        pltpu.VMEM_SHARED((_NP, _D), jnp.float32),
        pltpu.SemaphoreType.DMA,
        pltpu.SemaphoreType.DMA,
    ],
    compiler_params=pltpu.CompilerParams(needs_layout_passes=False),
)


def _pre_body(x_ref, w_ref, dv_ref, o_ref):
    h = jnp.dot(x_ref[...], w_ref[...], preferred_element_type=jnp.float32)
    o_ref[...] = dv_ref[...] * h


_pre_call = pl.pallas_call(
    _pre_body,
    grid=(_G,),
    in_specs=[
        pl.BlockSpec((_R, _D), lambda i: (i, 0)),
        pl.BlockSpec((_D, _D), lambda i: (0, 0)),
        pl.BlockSpec((_R, 1), lambda i: (i, 0)),
    ],
    out_specs=pl.BlockSpec((_R, _D), lambda i: (i, 0)),
    out_shape=jax.ShapeDtypeStruct((_NP, _D), jnp.float32),
)


def _mid_body(s_ref, hp_ref, dv_ref, b_ref, g_ref, be_ref, w_ref, o_ref):
    agg = s_ref[0] + s_ref[1] + hp_ref[...]
    a = dv_ref[...] * agg + b_ref[...]
    a = a * (g_ref[...] * _BN) + be_ref[...]
    a = jnp.maximum(a, 0.0)
    o_ref[...] = dv_ref[...] * jnp.dot(
        a, w_ref[...], preferred_element_type=jnp.float32)


_mid_call = pl.pallas_call(
    _mid_body,
    grid=(_G,),
    in_specs=[
        pl.BlockSpec((_NC, _R, _D), lambda i: (0, i, 0)),
        pl.BlockSpec((_R, _D), lambda i: (i, 0)),
        pl.BlockSpec((_R, 1), lambda i: (i, 0)),
        pl.BlockSpec((1, _D), lambda i: (0, 0)),
        pl.BlockSpec((1, _D), lambda i: (0, 0)),
        pl.BlockSpec((1, _D), lambda i: (0, 0)),
        pl.BlockSpec((_D, _D), lambda i: (0, 0)),
    ],
    out_specs=pl.BlockSpec((_R, _D), lambda i: (i, 0)),
    out_shape=jax.ShapeDtypeStruct((_NP, _D), jnp.float32),
)


def _fin_body(s_ref, hp_ref, dv_ref, b_ref, g_ref, be_ref, wp_ref, bp_ref,
              o_ref, acc_ref):
    i = pl.program_id(0)
    agg = s_ref[0] + s_ref[1] + hp_ref[...]
    a = dv_ref[...] * agg + b_ref[...]
    a = a * (g_ref[...] * _BN) + be_ref[...]
    a = jnp.maximum(a, 0.0)
    a = jnp.where(dv_ref[...] > 0.0, a, 0.0)
    part = jnp.sum(a, axis=0, keepdims=True)

    @pl.when(i == 0)
    def _():
        acc_ref[...] = part

    @pl.when(i > 0)
    def _():
        acc_ref[...] = acc_ref[...] + part

    @pl.when(i == pl.num_programs(0) - 1)
    def _():
        o_ref[...] = jnp.dot(
            acc_ref[...] * (1.0 / _N), wp_ref[...],
            preferred_element_type=jnp.float32) + bp_ref[...]


_fin_call = pl.pallas_call(
    _fin_body,
    grid=(_G,),
    in_specs=[
        pl.BlockSpec((_NC, _R, _D), lambda i: (0, i, 0)),
        pl.BlockSpec((_R, _D), lambda i: (i, 0)),
        pl.BlockSpec((_R, 1), lambda i: (i, 0)),
        pl.BlockSpec((1, _D), lambda i: (0, 0)),
        pl.BlockSpec((1, _D), lambda i: (0, 0)),
        pl.BlockSpec((1, _D), lambda i: (0, 0)),
        pl.BlockSpec((_D, _D), lambda i: (0, 0)),
        pl.BlockSpec((1, _D), lambda i: (0, 0)),
    ],
    out_specs=pl.BlockSpec((1, _D), lambda i: (0, 0)),
    out_shape=jax.ShapeDtypeStruct((1, _D), jnp.float32),
    scratch_shapes=[pltpu.VMEM((1, _D), jnp.float32)],
)


def kernel(x, edge_index, W1, b1, g1, be1, W2, b2, g2, be2, W3, b3, g3, be3,
           Wp, bp):
    src = edge_index[0]
    dst = edge_index[1]
    pad_cfg = ((0, 0), (0, _EPT2 - _EPT))
    src2 = jnp.pad(src.reshape(_NW, _EPT), pad_cfg,
                   constant_values=_NP - 1).reshape(-1)
    dst2 = jnp.pad(dst.reshape(_NW, _EPT), pad_cfg,
                   constant_values=_NP - 1).reshape(-1)

    dinv = _dinv_call(dst).reshape(_NP, 1)
    x_pad = jnp.concatenate(
        [x, jnp.zeros((_NP - _N, _D), jnp.float32)], axis=0)

    b1r, g1r, be1r = b1.reshape(1, _D), g1.reshape(1, _D), be1.reshape(1, _D)
    b2r, g2r, be2r = b2.reshape(1, _D), g2.reshape(1, _D), be2.reshape(1, _D)
    b3r, g3r, be3r = b3.reshape(1, _D), g3.reshape(1, _D), be3.reshape(1, _D)
    bpr = bp.reshape(1, _D)

    h1p = _pre_call(x_pad, W1, dinv)
    s1 = _agg_call(h1p, src2, dst2)
    h2p = _mid_call(s1, h1p, dinv, b1r, g1r, be1r, W2)
    s2 = _agg_call(h2p, src2, dst2)
    h3p = _mid_call(s2, h2p, dinv, b2r, g2r, be2r, W3)
    s3 = _agg_call(h3p, src2, dst2)
    out = _fin_call(s3, h3p, dinv, b3r, g3r, be3r, Wp, bpr)
    return out


# speedup vs baseline: 1.5642x; 1.5642x over previous
import jax
import jax.numpy as jnp
import numpy as np
from jax import lax
from jax.experimental import pallas as pl
from jax.experimental.pallas import tpu as pltpu
from jax.experimental.pallas import tpu_sc as plsc

_N = 10000
_E = 320000
_D = 128
_NC = 2
_NS = 16
_NW = _NC * _NS
_NP = 10240
_RPT = _NP // _NS
_EPT_DEG = _E // _NS
_EPT = _E // _NW
_C = 80
_NCHUNK = _EPT // _C
_R = 1024
_G = _NP // _R
_BN = float(1.0 / np.sqrt(1.0 + 1e-5))


def _dinv_body(dst_hbm, dinv_hbm, dstall, degbuf, tmp, stage_sh):
    c = lax.axis_index("c")
    s = lax.axis_index("s")
    zeros = jnp.zeros((16,), jnp.float32)
    ones = jnp.ones((16,), jnp.float32)

    @pl.when(c == 0)
    def _():
        def zero_step(i, carry):
            degbuf[pl.ds(i * 16, 16)] = zeros
            return carry

        lax.fori_loop(0, _NP // 16, zero_step, 0)

        pltpu.sync_copy(dst_hbm.at[pl.ds(s * _EPT_DEG, _EPT_DEG)], dstall)

        def scat_step(i, carry):
            idx = dstall[pl.ds(i * 16, 16)]
            plsc.addupdate_scatter(degbuf, [idx], ones)
            return carry

        lax.fori_loop(0, _EPT_DEG // 16, scat_step, 0)

        pltpu.sync_copy(degbuf, stage_sh.at[pl.ds(s * _NP, _NP)])
        plsc.subcore_barrier()

        for j in range(_NS):
            pltpu.sync_copy(stage_sh.at[pl.ds(j * _NP + s * _RPT, _RPT)],
                            tmp.at[pl.ds(j * _RPT, _RPT)])

        lane = lax.iota(jnp.int32, 16)

        def red_step(k, carry):
            acc = tmp[pl.ds(k * 16, 16)]
            for j in range(1, _NS):
                acc = acc + tmp[pl.ds(j * _RPT + k * 16, 16)]
            x = acc + 1.0
            ibits = lax.bitcast_convert_type(x, jnp.int32)
            ibits = jnp.int32(0x5F3759DF) - (ibits >> 1)
            y = lax.bitcast_convert_type(ibits, jnp.float32)
            for _ in range(3):
                y = y * (1.5 - 0.5 * x * y * y)
            node = s * _RPT + k * 16 + lane
            y = jnp.where(node < _N, y, 0.0)
            tmp[pl.ds(k * 16, 16)] = y
            return carry

        lax.fori_loop(0, _RPT // 16, red_step, 0)
        pltpu.sync_copy(tmp.at[pl.ds(0, _RPT)],
                        dinv_hbm.at[pl.ds(s * _RPT, _RPT)])


_dinv_call = pl.kernel(
    _dinv_body,
    out_type=jax.ShapeDtypeStruct((_NP,), jnp.float32),
    mesh=plsc.VectorSubcoreMesh(
        core_axis_name="c", subcore_axis_name="s", num_cores=_NC,
        num_subcores=_NS),
    scratch_types=[
        pltpu.VMEM((_EPT_DEG,), jnp.int32),
        pltpu.VMEM((_NP,), jnp.float32),
        pltpu.VMEM((_NS * _RPT,), jnp.float32),
        pltpu.VMEM_SHARED((_NS * _NP,), jnp.float32),
    ],
    compiler_params=pltpu.CompilerParams(needs_layout_passes=False),
)


def _agg_body(hp_hbm, src_hbm, dst_hbm, out_hbm,
              srcall, dstall, srcbuf0, dstbuf0, srcbuf1, dstbuf1,
              rows0, rows1, s_sh, gsem0, gsem1):
    c = lax.axis_index("c")
    s = lax.axis_index("s")
    wid = c * _NS + s
    base = wid * _EPT
    zeros = jnp.zeros((16,), jnp.float32)

    def zero_step(i, carry):
        rows0[i // 8, pl.ds((i % 8) * 16, 16)] = zeros
        return carry

    lax.fori_loop(0, _C * 8, zero_step, 0)
    for r in range(_RPT // _C):
        pltpu.sync_copy(rows0, s_sh.at[pl.ds(s * _RPT + r * _C, _C)])
    plsc.subcore_barrier()

    pltpu.sync_copy(src_hbm.at[pl.ds(base, _EPT)], srcall)
    pltpu.sync_copy(dst_hbm.at[pl.ds(base, _EPT)], dstall)

    def fill(k, sbuf, dbuf):
        for j in range(_C // 16):
            sbuf[pl.ds(j * 16, 16)] = srcall[pl.ds(k * _C + j * 16, 16)]
            dbuf[pl.ds(j * 16, 16)] = dstall[pl.ds(k * _C + j * 16, 16)]

    fill(0, srcbuf0, dstbuf0)
    pltpu.async_copy(hp_hbm.at[srcbuf0], rows0, gsem0)

    def pair_step(i, carry):
        k = 2 * i
        fill(k + 1, srcbuf1, dstbuf1)
        pltpu.async_copy(hp_hbm.at[srcbuf1], rows1, gsem1)
        pltpu.make_async_copy(hp_hbm.at[srcbuf0], rows0, gsem0).wait()
        pltpu.sync_copy(rows0, s_sh.at[dstbuf0], add=True)
        fill(k + 2, srcbuf0, dstbuf0)
        pltpu.async_copy(hp_hbm.at[srcbuf0], rows0, gsem0)
        pltpu.make_async_copy(hp_hbm.at[srcbuf1], rows1, gsem1).wait()
        pltpu.sync_copy(rows1, s_sh.at[dstbuf1], add=True)
        return carry

    lax.fori_loop(0, (_NCHUNK - 1) // 2, pair_step, 0)
    pltpu.make_async_copy(hp_hbm.at[srcbuf0], rows0, gsem0).wait()
    pltpu.sync_copy(rows0, s_sh.at[dstbuf0], add=True)
    plsc.subcore_barrier()

    for r in range(_RPT // _C):
        pltpu.sync_copy(s_sh.at[pl.ds(s * _RPT + r * _C, _C)], rows0)
        pltpu.sync_copy(rows0, out_hbm.at[c, pl.ds(s * _RPT + r * _C, _C)])


_agg_call = pl.kernel(
    _agg_body,
    out_type=jax.ShapeDtypeStruct((_NC, _NP, _D), jnp.float32),
    mesh=plsc.VectorSubcoreMesh(
        core_axis_name="c", subcore_axis_name="s", num_cores=_NC,
        num_subcores=_NS),
    scratch_types=[
        pltpu.VMEM((_EPT,), jnp.int32),
        pltpu.VMEM((_EPT,), jnp.int32),
        pltpu.VMEM((_C,), jnp.int32),
        pltpu.VMEM((_C,), jnp.int32),
        pltpu.VMEM((_C,), jnp.int32),
        pltpu.VMEM((_C,), jnp.int32),
        pltpu.VMEM((_C, _D), jnp.float32),
        pltpu.VMEM((_C, _D), jnp.float32),
        pltpu.VMEM_SHARED((_NP, _D), jnp.float32),
        pltpu.SemaphoreType.DMA,
        pltpu.SemaphoreType.DMA,
    ],
    compiler_params=pltpu.CompilerParams(needs_layout_passes=False),
)


def _pre_body(x_ref, w_ref, dv_ref, o_ref):
    h = jnp.dot(x_ref[...], w_ref[...], preferred_element_type=jnp.float32)
    o_ref[...] = dv_ref[...] * h


_pre_call = pl.pallas_call(
    _pre_body,
    grid=(_G,),
    in_specs=[
        pl.BlockSpec((_R, _D), lambda i: (i, 0)),
        pl.BlockSpec((_D, _D), lambda i: (0, 0)),
        pl.BlockSpec((_R, 1), lambda i: (i, 0)),
    ],
    out_specs=pl.BlockSpec((_R, _D), lambda i: (i, 0)),
    out_shape=jax.ShapeDtypeStruct((_NP, _D), jnp.float32),
)


def _mid_body(s_ref, hp_ref, dv_ref, b_ref, g_ref, be_ref, w_ref, o_ref):
    agg = s_ref[0] + s_ref[1] + hp_ref[...]
    a = dv_ref[...] * agg + b_ref[...]
    a = a * (g_ref[...] * _BN) + be_ref[...]
    a = jnp.maximum(a, 0.0)
    o_ref[...] = dv_ref[...] * jnp.dot(
        a, w_ref[...], preferred_element_type=jnp.float32)


_mid_call = pl.pallas_call(
    _mid_body,
    grid=(_G,),
    in_specs=[
        pl.BlockSpec((_NC, _R, _D), lambda i: (0, i, 0)),
        pl.BlockSpec((_R, _D), lambda i: (i, 0)),
        pl.BlockSpec((_R, 1), lambda i: (i, 0)),
        pl.BlockSpec((1, _D), lambda i: (0, 0)),
        pl.BlockSpec((1, _D), lambda i: (0, 0)),
        pl.BlockSpec((1, _D), lambda i: (0, 0)),
        pl.BlockSpec((_D, _D), lambda i: (0, 0)),
    ],
    out_specs=pl.BlockSpec((_R, _D), lambda i: (i, 0)),
    out_shape=jax.ShapeDtypeStruct((_NP, _D), jnp.float32),
)


def _fin_body(s_ref, hp_ref, dv_ref, b_ref, g_ref, be_ref, wp_ref, bp_ref,
              o_ref, acc_ref):
    i = pl.program_id(0)
    agg = s_ref[0] + s_ref[1] + hp_ref[...]
    a = dv_ref[...] * agg + b_ref[...]
    a = a * (g_ref[...] * _BN) + be_ref[...]
    a = jnp.maximum(a, 0.0)
    a = jnp.where(dv_ref[...] > 0.0, a, 0.0)
    part = jnp.sum(a, axis=0, keepdims=True)

    @pl.when(i == 0)
    def _():
        acc_ref[...] = part

    @pl.when(i > 0)
    def _():
        acc_ref[...] = acc_ref[...] + part

    @pl.when(i == pl.num_programs(0) - 1)
    def _():
        o_ref[...] = jnp.dot(
            acc_ref[...] * (1.0 / _N), wp_ref[...],
            preferred_element_type=jnp.float32) + bp_ref[...]


_fin_call = pl.pallas_call(
    _fin_body,
    grid=(_G,),
    in_specs=[
        pl.BlockSpec((_NC, _R, _D), lambda i: (0, i, 0)),
        pl.BlockSpec((_R, _D), lambda i: (i, 0)),
        pl.BlockSpec((_R, 1), lambda i: (i, 0)),
        pl.BlockSpec((1, _D), lambda i: (0, 0)),
        pl.BlockSpec((1, _D), lambda i: (0, 0)),
        pl.BlockSpec((1, _D), lambda i: (0, 0)),
        pl.BlockSpec((_D, _D), lambda i: (0, 0)),
        pl.BlockSpec((1, _D), lambda i: (0, 0)),
    ],
    out_specs=pl.BlockSpec((1, _D), lambda i: (0, 0)),
    out_shape=jax.ShapeDtypeStruct((1, _D), jnp.float32),
    scratch_shapes=[pltpu.VMEM((1, _D), jnp.float32)],
)


def kernel(x, edge_index, W1, b1, g1, be1, W2, b2, g2, be2, W3, b3, g3, be3,
           Wp, bp):
    src = edge_index[0]
    dst = edge_index[1]

    dinv = _dinv_call(dst).reshape(_NP, 1)
    x_pad = jnp.concatenate(
        [x, jnp.zeros((_NP - _N, _D), jnp.float32)], axis=0)

    b1r, g1r, be1r = b1.reshape(1, _D), g1.reshape(1, _D), be1.reshape(1, _D)
    b2r, g2r, be2r = b2.reshape(1, _D), g2.reshape(1, _D), be2.reshape(1, _D)
    b3r, g3r, be3r = b3.reshape(1, _D), g3.reshape(1, _D), be3.reshape(1, _D)
    bpr = bp.reshape(1, _D)

    h1p = _pre_call(x_pad, W1, dinv)
    s1 = _agg_call(h1p, src, dst)
    h2p = _mid_call(s1, h1p, dinv, b1r, g1r, be1r, W2)
    s2 = _agg_call(h2p, src, dst)
    h3p = _mid_call(s2, h2p, dinv, b2r, g2r, be2r, W3)
    s3 = _agg_call(h3p, src, dst)
    out = _fin_call(s3, h3p, dinv, b3r, g3r, be3r, Wp, bpr)
    return out
